# baseline (speedup 1.0000x reference)
import functools

import numpy as np
import jax
import jax.numpy as jnp
from jax import lax
from jax.experimental import pallas as pl
from jax.experimental.pallas import tpu as pltpu
from jax.experimental.pallas import tpu_sc as plsc

NUM_VIEW = 300
NUM_VIEW_PAD = 384
FEAT_DIM = 256
NUM_SAMPLE = 1024
EPS = 1e-5
THRESH = 0.09

_INTERPRET = False


def _template_views(n):
    phi = (np.sqrt(5.0) - 1.0) / 2.0
    i = np.arange(n, dtype=np.float64)
    zi = (2.0 * i + 1.0) / n - 1.0
    r = np.sqrt(np.clip(1.0 - zi * zi, 0.0, None))
    xi = r * np.cos(2.0 * i * np.pi * phi)
    yi = r * np.sin(2.0 * i * np.pi * phi)
    return np.stack([xi, yi, zi], axis=1).astype(np.float32)


def _score_kernel(f_ref, w1_ref, b1_ref, g1_ref, be1_ref, m1_ref,
                  v1_ref, w2_ref, b2_ref, score_ref, featT_ref):
    x = f_ref[0]
    h = jnp.dot(w1_ref[...], x, preferred_element_type=jnp.float32) + b1_ref[...]
    h = (h - m1_ref[...]) / jnp.sqrt(v1_ref[...] + EPS) * g1_ref[...] + be1_ref[...]
    h = jnp.maximum(h, 0.0)
    s = jnp.dot(w2_ref[...], h, preferred_element_type=jnp.float32) + b2_ref[0, 0]
    score_ref[0] = s
    featT_ref[0] = x.T


def _graspness_head(seed_features, p):
    Bb, C, N = seed_features.shape
    BLK = 2048
    grid = (Bb, N // BLK)
    full = lambda shp: pl.BlockSpec(shp, lambda b, j: (0,) * len(shp))
    score, featT = pl.pallas_call(
        _score_kernel,
        grid=grid,
        in_specs=[
            pl.BlockSpec((1, C, BLK), lambda b, j: (b, 0, j)),
            full((C, C)), full((C, 1)), full((C, 1)), full((C, 1)),
            full((C, 1)), full((C, 1)), full((1, C)), full((1, 1)),
        ],
        out_specs=[
            pl.BlockSpec((1, 1, BLK), lambda b, j: (b, 0, j)),
            pl.BlockSpec((1, BLK, C), lambda b, j: (b, j, 0)),
        ],
        out_shape=[
            jax.ShapeDtypeStruct((Bb, 1, N), jnp.float32),
            jax.ShapeDtypeStruct((Bb, N, C), jnp.float32),
        ],
        interpret=_INTERPRET,
    )(seed_features,
      p['gh_w1'], p['gh_b1'].reshape(C, 1), p['gh_g1'].reshape(C, 1),
      p['gh_be1'].reshape(C, 1), p['gh_m1'].reshape(C, 1),
      p['gh_v1'].reshape(C, 1), p['gh_w2'].reshape(1, C),
      p['gh_b2'].reshape(1, 1))
    return score, featT


def _fps_kernel(xyz_ref, score_ref, inds_ref, out4_ref):
    Bb = xyz_ref.shape[0]
    R = xyz_ref.shape[2]
    Cc = xyz_ref.shape[3]
    N = R * Cc
    BIG = jnp.int32(N)
    idx2 = (lax.broadcasted_iota(jnp.int32, (R, Cc), 0) * Cc
            + lax.broadcasted_iota(jnp.int32, (R, Cc), 1))
    lane = lax.broadcasted_iota(jnp.int32, (1, Cc), 1)

    dist0s = []
    far0s = []
    for b in range(Bb):
        s = score_ref[b]
        m = s > THRESH
        anym = jnp.any(m)
        m = jnp.logical_or(m, jnp.logical_not(anym))
        dist0s.append(jnp.where(m, jnp.float32(1e10), -jnp.inf))
        far0s.append(jnp.min(jnp.where(m, idx2, BIG), keepdims=True,
                             axis=(0, 1)))

    def body(i, carry):
        dists = carry[:Bb]
        fars = carry[Bb:]
        ndists = []
        nfars = []
        for b in range(Bb):
            f11 = fars[b]
            rb = f11[0, 0] // Cc
            cb11 = jnp.remainder(f11, Cc)
            inds_ref[pl.ds(i, 1), pl.ds(b, 1)] = f11 + b * N
            selc = lane == cb11
            xr = xyz_ref[b, 0, pl.ds(rb, 1), :]
            yr = xyz_ref[b, 1, pl.ds(rb, 1), :]
            zr = xyz_ref[b, 2, pl.ds(rb, 1), :]
            sr = score_ref[b, pl.ds(rb, 1), :]
            fx = jnp.sum(jnp.where(selc, xr, 0.0), axis=1, keepdims=True)
            fy = jnp.sum(jnp.where(selc, yr, 0.0), axis=1, keepdims=True)
            fz = jnp.sum(jnp.where(selc, zr, 0.0), axis=1, keepdims=True)
            fs = jnp.sum(jnp.where(selc, sr, 0.0), axis=1, keepdims=True)
            row = jnp.concatenate([fx, fy, fz, fs], axis=1)
            out4_ref[pl.ds(i, 1), pl.ds(4 * b, 4)] = row
            dx = xyz_ref[b, 0] - fx
            dy = xyz_ref[b, 1] - fy
            dz = xyz_ref[b, 2] - fz
            d = dx * dx + dy * dy + dz * dz
            nd = jnp.minimum(dists[b], d)
            mx = jnp.max(nd, keepdims=True, axis=(0, 1))
            nfars.append(jnp.min(jnp.where(nd == mx, idx2, BIG),
                                 keepdims=True, axis=(0, 1)))
            ndists.append(nd)
        return tuple(ndists) + tuple(nfars)

    lax.fori_loop(0, NUM_SAMPLE, body, tuple(dist0s) + tuple(far0s))


def _fps(xyzT, score):
    Bb, _, N = xyzT.shape
    R = 128
    Cc = N // R
    xyz4 = xyzT.reshape(Bb, 3, R, Cc)
    score4 = score.reshape(Bb, R, Cc)
    full = lambda shp: pl.BlockSpec(shp, lambda: (0,) * len(shp))
    inds, out4 = pl.pallas_call(
        _fps_kernel,
        in_specs=[full((Bb, 3, R, Cc)), full((Bb, R, Cc))],
        out_specs=[full((NUM_SAMPLE, 8)), full((NUM_SAMPLE, 4 * Bb))],
        out_shape=[
            jax.ShapeDtypeStruct((NUM_SAMPLE, 8), jnp.int32),
            jax.ShapeDtypeStruct((NUM_SAMPLE, 4 * Bb), jnp.float32),
        ],
        interpret=_INTERPRET,
    )(xyz4, score4)
    inds_g = inds[:, :Bb].T.reshape(-1)
    pk = out4.reshape(NUM_SAMPLE, Bb, 4).transpose(1, 0, 2)
    return inds_g, pk[:, :, :3], pk[:, :, 3]


def _gather_rows(featT_flat, inds_flat):
    M, D = featT_flat.shape
    T = inds_flat.shape[0]
    NW = 32
    per = T // NW
    mesh = plsc.VectorSubcoreMesh(core_axis_name="c", subcore_axis_name="s")

    @functools.partial(
        pl.kernel, mesh=mesh,
        out_type=jax.ShapeDtypeStruct((T, D), jnp.float32),
        scratch_types=[
            pltpu.VMEM((per,), jnp.int32),
            pltpu.VMEM((per, D), jnp.float32),
            pltpu.SemaphoreType.DMA,
        ],
    )
    def gk(inds_hbm, tab_hbm, out_hbm, idx_v, rows_v, sem):
        wid = lax.axis_index("s") * 2 + lax.axis_index("c")
        base = wid * per
        pltpu.sync_copy(inds_hbm.at[pl.ds(base, per)], idx_v)
        pltpu.async_copy(tab_hbm.at[idx_v], rows_v, sem).wait()
        pltpu.sync_copy(rows_v, out_hbm.at[pl.ds(base, per)])

    return gk(inds_flat, featT_flat)


def _view_kernel(f_ref, w1_ref, b1_ref, g1_ref, be1_ref, m1_ref, v1_ref,
                 w2_ref, b2_ref, g2_ref, be2_ref, m2_ref, v2_ref,
                 w3_ref, b3_ref, tvx_ref, tvy_ref, tvz_ref,
                 vs_ref, tvs_ref, vp_ref, rot_ref):
    F = f_ref[0]
    h = jnp.dot(w1_ref[...], F, preferred_element_type=jnp.float32) + b1_ref[...]
    h = (h - m1_ref[...]) / jnp.sqrt(v1_ref[...] + EPS) * g1_ref[...] + be1_ref[...]
    h = jnp.maximum(h, 0.0)
    h = jnp.dot(w2_ref[...], h, preferred_element_type=jnp.float32) + b2_ref[...]
    h = (h - m2_ref[...]) / jnp.sqrt(v2_ref[...] + EPS) * g2_ref[...] + be2_ref[...]
    h = jnp.maximum(h, 0.0)
    h3 = jnp.dot(w3_ref[...], h, preferred_element_type=jnp.float32) + b3_ref[...]
    vs_ref[0] = h3

    V, NS = h3.shape
    rowidx = lax.broadcasted_iota(jnp.int32, (V, NS), 0)
    h3m = jnp.where(rowidx < NUM_VIEW, h3, -jnp.inf)
    mx = jnp.max(h3m, axis=0, keepdims=True)
    tvs_ref[0] = mx
    am = jnp.min(jnp.where(h3m == mx, rowidx, jnp.int32(V)), axis=0,
                 keepdims=True)
    sel = rowidx == am
    vpx = jnp.sum(jnp.where(sel, tvx_ref[...], 0.0), axis=0, keepdims=True)
    vpy = jnp.sum(jnp.where(sel, tvy_ref[...], 0.0), axis=0, keepdims=True)
    vpz = jnp.sum(jnp.where(sel, tvz_ref[...], 0.0), axis=0, keepdims=True)
    vp_ref[0] = jnp.concatenate([vpx, vpy, vpz], axis=0)

    tx, ty, tz = -vpx, -vpy, -vpz
    ay0r, ay1r = -ty, tx
    ny = jnp.sqrt(ay0r * ay0r + ay1r * ay1r + 0.0)
    mz = ny == 0.0
    ay0 = jnp.where(mz, 0.0, ay0r)
    ay1 = jnp.where(mz, 1.0, ay1r)
    ay2 = jnp.zeros_like(ay0)
    nx = jnp.sqrt(tx * tx + ty * ty + tz * tz)
    ax0, ax1, ax2 = tx / nx, ty / nx, tz / nx
    nyn = jnp.sqrt(ay0 * ay0 + ay1 * ay1 + ay2 * ay2)
    ay0, ay1, ay2 = ay0 / nyn, ay1 / nyn, ay2 / nyn
    az0 = ax1 * ay2 - ax2 * ay1
    az1 = ax2 * ay0 - ax0 * ay2
    az2 = ax0 * ay1 - ax1 * ay0
    rot_ref[0] = jnp.concatenate(
        [ax0, ay0, az0, ax1, ay1, az1, ax2, ay2, az2], axis=0)


def _view_head(gfeat, p, tv_pad):
    Bb, C, NS = gfeat.shape
    V = NUM_VIEW_PAD
    full = lambda shp: pl.BlockSpec(shp, lambda b: (0,) * len(shp))

    def padw(w):
        return jnp.concatenate(
            [w, jnp.zeros((V - NUM_VIEW, w.shape[1]), w.dtype)], axis=0)

    def padv(v, fill=0.0):
        return jnp.concatenate(
            [v, jnp.full((V - NUM_VIEW,), fill, v.dtype)]).reshape(V, 1)

    vs, tvs, vp, rot = pl.pallas_call(
        _view_kernel,
        grid=(Bb,),
        in_specs=[
            pl.BlockSpec((1, C, NS), lambda b: (b, 0, 0)),
            full((C, C)), full((C, 1)), full((C, 1)), full((C, 1)),
            full((C, 1)), full((C, 1)),
            full((V, C)), full((V, 1)), full((V, 1)), full((V, 1)),
            full((V, 1)), full((V, 1)),
            full((V, V)), full((V, 1)),
            full((V, 1)), full((V, 1)), full((V, 1)),
        ],
        out_specs=[
            pl.BlockSpec((1, V, NS), lambda b: (b, 0, 0)),
            pl.BlockSpec((1, 1, NS), lambda b: (b, 0, 0)),
            pl.BlockSpec((1, 3, NS), lambda b: (b, 0, 0)),
            pl.BlockSpec((1, 9, NS), lambda b: (b, 0, 0)),
        ],
        out_shape=[
            jax.ShapeDtypeStruct((Bb, V, NS), jnp.float32),
            jax.ShapeDtypeStruct((Bb, 1, NS), jnp.float32),
            jax.ShapeDtypeStruct((Bb, 3, NS), jnp.float32),
            jax.ShapeDtypeStruct((Bb, 9, NS), jnp.float32),
        ],
        interpret=_INTERPRET,
    )(gfeat,
      p['w1'], p['b1'].reshape(C, 1), p['g1'].reshape(C, 1),
      p['be1'].reshape(C, 1), p['m1'].reshape(C, 1), p['v1'].reshape(C, 1),
      padw(p['w2']), padv(p['b2']), padv(p['g2']), padv(p['be2']),
      padv(p['m2']), padv(p['v2'], 1.0),
      jnp.pad(padw(p['w3']), ((0, 0), (0, V - NUM_VIEW))), padv(p['b3']),
      tv_pad[:, 0].reshape(V, 1), tv_pad[:, 1].reshape(V, 1),
      tv_pad[:, 2].reshape(V, 1))
    return vs, tvs, vp, rot


def kernel(seed_xyz, seed_features, params):
    Bb, N, _ = seed_xyz.shape
    C = seed_features.shape[1]

    xyzT = seed_xyz.transpose(0, 2, 1)
    score3, featT = _graspness_head(seed_features, params)
    graspness_score = score3[:, 0, :]

    inds_g, graspable_xyz, fp2_graspness = _fps(xyzT, graspness_score)

    gflat = _gather_rows(featT.reshape(Bb * N, C), inds_g)
    graspable_features = gflat.reshape(Bb, NUM_SAMPLE, C).transpose(0, 2, 1)

    tv = jnp.asarray(np.concatenate(
        [_template_views(NUM_VIEW),
         np.zeros((NUM_VIEW_PAD - NUM_VIEW, 3), np.float32)], axis=0))
    vs, tvs, vp, rot = _view_head(graspable_features, params, tv)

    view_score = vs[:, :NUM_VIEW, :].transpose(0, 2, 1)
    top_view_scores = tvs[:, 0, :]
    vp_xyz = vp.transpose(0, 2, 1)
    vp_rot = rot.reshape(Bb, 3, 3, NUM_SAMPLE).transpose(0, 3, 1, 2)

    return (graspness_score, graspable_xyz, graspable_features,
            fp2_graspness, view_score, top_view_scores, vp_xyz, vp_rot)


# speedup vs baseline: 2.8697x; 2.8697x over previous
import functools

import numpy as np
import jax
import jax.numpy as jnp
from jax import lax
from jax.experimental import pallas as pl
from jax.experimental.pallas import tpu as pltpu
from jax.experimental.pallas import tpu_sc as plsc

NUM_VIEW = 300
NUM_VIEW_PAD = 384
FEAT_DIM = 256
NUM_SAMPLE = 1024
EPS = 1e-5
THRESH = 0.09

_INTERPRET = False


def _template_views(n):
    phi = (np.sqrt(5.0) - 1.0) / 2.0
    i = np.arange(n, dtype=np.float64)
    zi = (2.0 * i + 1.0) / n - 1.0
    r = np.sqrt(np.clip(1.0 - zi * zi, 0.0, None))
    xi = r * np.cos(2.0 * i * np.pi * phi)
    yi = r * np.sin(2.0 * i * np.pi * phi)
    return np.stack([xi, yi, zi], axis=1).astype(np.float32)


def _score_kernel(f_ref, xyzT_ref, w1_ref, b1_ref, g1_ref, be1_ref, m1_ref,
                  v1_ref, w2_ref, b2_ref, score_ref, featT_ref):
    x = f_ref[0]
    h = jnp.dot(w1_ref[...], x, preferred_element_type=jnp.float32) + b1_ref[...]
    h = (h - m1_ref[...]) / jnp.sqrt(v1_ref[...] + EPS) * g1_ref[...] + be1_ref[...]
    h = jnp.maximum(h, 0.0)
    s = jnp.dot(w2_ref[...], h, preferred_element_type=jnp.float32) + b2_ref[0, 0]
    score_ref[0] = s
    wide = jnp.concatenate(
        [x, xyzT_ref[0], s, jnp.zeros((124, x.shape[1]), jnp.float32)], axis=0)
    featT_ref[0] = wide.T


def _graspness_head(seed_features, xyzT, p):
    Bb, C, N = seed_features.shape
    W = C + 128
    BLK = 2048
    grid = (Bb, N // BLK)
    full = lambda shp: pl.BlockSpec(shp, lambda b, j: (0,) * len(shp))
    score, featT = pl.pallas_call(
        _score_kernel,
        grid=grid,
        in_specs=[
            pl.BlockSpec((1, C, BLK), lambda b, j: (b, 0, j)),
            pl.BlockSpec((1, 3, BLK), lambda b, j: (b, 0, j)),
            full((C, C)), full((C, 1)), full((C, 1)), full((C, 1)),
            full((C, 1)), full((C, 1)), full((1, C)), full((1, 1)),
        ],
        out_specs=[
            pl.BlockSpec((1, 1, BLK), lambda b, j: (b, 0, j)),
            pl.BlockSpec((1, BLK, W), lambda b, j: (b, j, 0)),
        ],
        out_shape=[
            jax.ShapeDtypeStruct((Bb, 1, N), jnp.float32),
            jax.ShapeDtypeStruct((Bb, N, W), jnp.float32),
        ],
        interpret=_INTERPRET,
    )(seed_features, xyzT,
      p['gh_w1'], p['gh_b1'].reshape(C, 1), p['gh_g1'].reshape(C, 1),
      p['gh_be1'].reshape(C, 1), p['gh_m1'].reshape(C, 1),
      p['gh_v1'].reshape(C, 1), p['gh_w2'].reshape(1, C),
      p['gh_b2'].reshape(1, 1))
    return score, featT


def _fps_kernel(xyz_ref, score_ref, inds_ref):
    Bb = xyz_ref.shape[0]
    R = xyz_ref.shape[2]
    Cc = xyz_ref.shape[3]
    N = R * Cc
    x = xyz_ref[:, 0]
    y = xyz_ref[:, 1]
    z = xyz_ref[:, 2]
    s = score_ref[...]
    idx = (lax.broadcasted_iota(jnp.int32, (Bb, R, Cc), 1) * Cc
           + lax.broadcasted_iota(jnp.int32, (Bb, R, Cc), 2))
    m = s > THRESH
    anym = jnp.any(m, axis=(1, 2), keepdims=True)
    m = jnp.logical_or(m, jnp.logical_not(anym))
    dist0 = jnp.where(m, jnp.float32(1e10), -jnp.inf)
    BIG = jnp.int32(N)
    far0 = jnp.min(jnp.where(m, idx, BIG), axis=(1, 2))
    bofs = lax.iota(jnp.int32, Bb) * N

    def body(i, carry):
        dist, far = carry
        sel = idx == far[:, None, None]
        fx = jnp.sum(jnp.where(sel, x, 0.0), axis=(1, 2))
        fy = jnp.sum(jnp.where(sel, y, 0.0), axis=(1, 2))
        fz = jnp.sum(jnp.where(sel, z, 0.0), axis=(1, 2))
        inds_ref[pl.ds(i, 1), :] = jnp.concatenate(
            [far + bofs, jnp.zeros((Bb,), jnp.int32)])[None, :]
        dx = x - fx[:, None, None]
        dy = y - fy[:, None, None]
        dz = z - fz[:, None, None]
        d = dx * dx + dy * dy + dz * dz
        dist = jnp.minimum(dist, d)
        mx = jnp.max(dist, axis=(1, 2))
        far = jnp.min(
            jnp.where(dist == mx[:, None, None], idx, BIG), axis=(1, 2))
        return dist, far

    lax.fori_loop(0, NUM_SAMPLE, body, (dist0, far0))


def _fps(xyzT, score):
    Bb = xyzT.shape[0]
    N = xyzT.shape[2]
    R = 128
    Cc = N // R
    xyz4 = xyzT.reshape(Bb, 3, R, Cc)
    score4 = score.reshape(Bb, R, Cc)
    full = lambda shp: pl.BlockSpec(shp, lambda: (0,) * len(shp))
    inds = pl.pallas_call(
        _fps_kernel,
        in_specs=[full((Bb, 3, R, Cc)), full((Bb, R, Cc))],
        out_specs=full((NUM_SAMPLE, 2 * Bb)),
        out_shape=jax.ShapeDtypeStruct((NUM_SAMPLE, 2 * Bb), jnp.int32),
        interpret=_INTERPRET,
    )(xyz4, score4)
    return inds[:, :Bb].T.reshape(-1)


def _gather_rows(featT_flat, inds_flat):
    M, D = featT_flat.shape
    T = inds_flat.shape[0]
    NW = 32
    per = T // NW
    mesh = plsc.VectorSubcoreMesh(core_axis_name="c", subcore_axis_name="s")

    @functools.partial(
        pl.kernel, mesh=mesh,
        out_type=jax.ShapeDtypeStruct((T, D), jnp.float32),
        scratch_types=[
            pltpu.VMEM((per,), jnp.int32),
            pltpu.VMEM((per, D), jnp.float32),
            pltpu.SemaphoreType.DMA,
        ],
    )
    def gk(inds_hbm, tab_hbm, out_hbm, idx_v, rows_v, sem):
        wid = lax.axis_index("s") * 2 + lax.axis_index("c")
        base = wid * per
        pltpu.sync_copy(inds_hbm.at[pl.ds(base, per)], idx_v)
        pltpu.async_copy(tab_hbm.at[idx_v], rows_v, sem).wait()
        pltpu.sync_copy(rows_v, out_hbm.at[pl.ds(base, per)])

    return gk(inds_flat, featT_flat)


def _view_kernel(f_ref, w1_ref, b1_ref, g1_ref, be1_ref, m1_ref, v1_ref,
                 w2_ref, b2_ref, g2_ref, be2_ref, m2_ref, v2_ref,
                 w3_ref, b3_ref, tvx_ref, tvy_ref, tvz_ref,
                 vs_ref, tvs_ref, vp_ref, rot_ref):
    F = f_ref[0]
    h = jnp.dot(w1_ref[...], F, preferred_element_type=jnp.float32) + b1_ref[...]
    h = (h - m1_ref[...]) / jnp.sqrt(v1_ref[...] + EPS) * g1_ref[...] + be1_ref[...]
    h = jnp.maximum(h, 0.0)
    h = jnp.dot(w2_ref[...], h, preferred_element_type=jnp.float32) + b2_ref[...]
    h = (h - m2_ref[...]) / jnp.sqrt(v2_ref[...] + EPS) * g2_ref[...] + be2_ref[...]
    h = jnp.maximum(h, 0.0)
    h3 = jnp.dot(w3_ref[...], h, preferred_element_type=jnp.float32) + b3_ref[...]
    vs_ref[0] = h3

    V, NS = h3.shape
    rowidx = lax.broadcasted_iota(jnp.int32, (V, NS), 0)
    h3m = jnp.where(rowidx < NUM_VIEW, h3, -jnp.inf)
    mx = jnp.max(h3m, axis=0, keepdims=True)
    tvs_ref[0] = mx
    am = jnp.min(jnp.where(h3m == mx, rowidx, jnp.int32(V)), axis=0,
                 keepdims=True)
    sel = rowidx == am
    vpx = jnp.sum(jnp.where(sel, tvx_ref[...], 0.0), axis=0, keepdims=True)
    vpy = jnp.sum(jnp.where(sel, tvy_ref[...], 0.0), axis=0, keepdims=True)
    vpz = jnp.sum(jnp.where(sel, tvz_ref[...], 0.0), axis=0, keepdims=True)
    vp_ref[0] = jnp.concatenate([vpx, vpy, vpz], axis=0)

    tx, ty, tz = -vpx, -vpy, -vpz
    ay0r, ay1r = -ty, tx
    ny = jnp.sqrt(ay0r * ay0r + ay1r * ay1r + 0.0)
    mz = ny == 0.0
    ay0 = jnp.where(mz, 0.0, ay0r)
    ay1 = jnp.where(mz, 1.0, ay1r)
    ay2 = jnp.zeros_like(ay0)
    nx = jnp.sqrt(tx * tx + ty * ty + tz * tz)
    ax0, ax1, ax2 = tx / nx, ty / nx, tz / nx
    nyn = jnp.sqrt(ay0 * ay0 + ay1 * ay1 + ay2 * ay2)
    ay0, ay1, ay2 = ay0 / nyn, ay1 / nyn, ay2 / nyn
    az0 = ax1 * ay2 - ax2 * ay1
    az1 = ax2 * ay0 - ax0 * ay2
    az2 = ax0 * ay1 - ax1 * ay0
    rot_ref[0] = jnp.concatenate(
        [ax0, ay0, az0, ax1, ay1, az1, ax2, ay2, az2], axis=0)


def _view_head(gfeat, p, tv_pad):
    Bb, C, NS = gfeat.shape
    V = NUM_VIEW_PAD
    full = lambda shp: pl.BlockSpec(shp, lambda b: (0,) * len(shp))

    def padw(w):
        return jnp.concatenate(
            [w, jnp.zeros((V - NUM_VIEW, w.shape[1]), w.dtype)], axis=0)

    def padv(v, fill=0.0):
        return jnp.concatenate(
            [v, jnp.full((V - NUM_VIEW,), fill, v.dtype)]).reshape(V, 1)

    vs, tvs, vp, rot = pl.pallas_call(
        _view_kernel,
        grid=(Bb,),
        in_specs=[
            pl.BlockSpec((1, C, NS), lambda b: (b, 0, 0)),
            full((C, C)), full((C, 1)), full((C, 1)), full((C, 1)),
            full((C, 1)), full((C, 1)),
            full((V, C)), full((V, 1)), full((V, 1)), full((V, 1)),
            full((V, 1)), full((V, 1)),
            full((V, V)), full((V, 1)),
            full((V, 1)), full((V, 1)), full((V, 1)),
        ],
        out_specs=[
            pl.BlockSpec((1, V, NS), lambda b: (b, 0, 0)),
            pl.BlockSpec((1, 1, NS), lambda b: (b, 0, 0)),
            pl.BlockSpec((1, 3, NS), lambda b: (b, 0, 0)),
            pl.BlockSpec((1, 9, NS), lambda b: (b, 0, 0)),
        ],
        out_shape=[
            jax.ShapeDtypeStruct((Bb, V, NS), jnp.float32),
            jax.ShapeDtypeStruct((Bb, 1, NS), jnp.float32),
            jax.ShapeDtypeStruct((Bb, 3, NS), jnp.float32),
            jax.ShapeDtypeStruct((Bb, 9, NS), jnp.float32),
        ],
        interpret=_INTERPRET,
    )(gfeat,
      p['w1'], p['b1'].reshape(C, 1), p['g1'].reshape(C, 1),
      p['be1'].reshape(C, 1), p['m1'].reshape(C, 1), p['v1'].reshape(C, 1),
      padw(p['w2']), padv(p['b2']), padv(p['g2']), padv(p['be2']),
      padv(p['m2']), padv(p['v2'], 1.0),
      jnp.pad(padw(p['w3']), ((0, 0), (0, V - NUM_VIEW))), padv(p['b3']),
      tv_pad[:, 0].reshape(V, 1), tv_pad[:, 1].reshape(V, 1),
      tv_pad[:, 2].reshape(V, 1))
    return vs, tvs, vp, rot


def kernel(seed_xyz, seed_features, params):
    Bb, N, _ = seed_xyz.shape
    C = seed_features.shape[1]

    xyzT = seed_xyz.transpose(0, 2, 1)
    score3, featT = _graspness_head(seed_features, xyzT, params)
    graspness_score = score3[:, 0, :]

    inds_g = _fps(xyzT, graspness_score)

    gflat = _gather_rows(featT.reshape(Bb * N, C + 128), inds_g)
    graspable_features = gflat[:, :C].reshape(Bb, NUM_SAMPLE, C).transpose(0, 2, 1)
    graspable_xyz = gflat[:, C:C + 3].reshape(Bb, NUM_SAMPLE, 3)
    fp2_graspness = gflat[:, C + 3].reshape(Bb, NUM_SAMPLE)

    tv = jnp.asarray(np.concatenate(
        [_template_views(NUM_VIEW),
         np.zeros((NUM_VIEW_PAD - NUM_VIEW, 3), np.float32)], axis=0))
    vs, tvs, vp, rot = _view_head(graspable_features, params, tv)

    view_score = vs[:, :NUM_VIEW, :].transpose(0, 2, 1)
    top_view_scores = tvs[:, 0, :]
    vp_xyz = vp.transpose(0, 2, 1)
    vp_rot = rot.reshape(Bb, 3, 3, NUM_SAMPLE).transpose(0, 3, 1, 2)

    return (graspness_score, graspable_xyz, graspable_features,
            fp2_graspness, view_score, top_view_scores, vp_xyz, vp_rot)


# speedup vs baseline: 3.1708x; 1.1049x over previous
import functools

import numpy as np
import jax
import jax.numpy as jnp
from jax import lax
from jax.experimental import pallas as pl
from jax.experimental.pallas import tpu as pltpu
from jax.experimental.pallas import tpu_sc as plsc

NUM_VIEW = 300
NUM_VIEW_PAD = 384
FEAT_DIM = 256
NUM_SAMPLE = 1024
EPS = 1e-5
THRESH = 0.09

_INTERPRET = False


def _template_views(n):
    phi = (np.sqrt(5.0) - 1.0) / 2.0
    i = np.arange(n, dtype=np.float64)
    zi = (2.0 * i + 1.0) / n - 1.0
    r = np.sqrt(np.clip(1.0 - zi * zi, 0.0, None))
    xi = r * np.cos(2.0 * i * np.pi * phi)
    yi = r * np.sin(2.0 * i * np.pi * phi)
    return np.stack([xi, yi, zi], axis=1).astype(np.float32)


def _score_kernel(f_ref, xyzT_ref, w1_ref, b1_ref, g1_ref, be1_ref, m1_ref,
                  v1_ref, w2_ref, b2_ref, score_ref, featT_ref):
    x = f_ref[0]
    h = jnp.dot(w1_ref[...], x, preferred_element_type=jnp.float32) + b1_ref[...]
    h = (h - m1_ref[...]) / jnp.sqrt(v1_ref[...] + EPS) * g1_ref[...] + be1_ref[...]
    h = jnp.maximum(h, 0.0)
    s = jnp.dot(w2_ref[...], h, preferred_element_type=jnp.float32) + b2_ref[0, 0]
    score_ref[0] = s
    wide = jnp.concatenate(
        [x, xyzT_ref[0], s, jnp.zeros((124, x.shape[1]), jnp.float32)], axis=0)
    featT_ref[0] = wide.T


def _graspness_head(seed_features, xyzT, p):
    Bb, C, N = seed_features.shape
    W = C + 128
    BLK = 2048
    grid = (Bb, N // BLK)
    full = lambda shp: pl.BlockSpec(shp, lambda b, j: (0,) * len(shp))
    score, featT = pl.pallas_call(
        _score_kernel,
        grid=grid,
        in_specs=[
            pl.BlockSpec((1, C, BLK), lambda b, j: (b, 0, j)),
            pl.BlockSpec((1, 3, BLK), lambda b, j: (b, 0, j)),
            full((C, C)), full((C, 1)), full((C, 1)), full((C, 1)),
            full((C, 1)), full((C, 1)), full((1, C)), full((1, 1)),
        ],
        out_specs=[
            pl.BlockSpec((1, 1, BLK), lambda b, j: (b, 0, j)),
            pl.BlockSpec((1, BLK, W), lambda b, j: (b, j, 0)),
        ],
        out_shape=[
            jax.ShapeDtypeStruct((Bb, 1, N), jnp.float32),
            jax.ShapeDtypeStruct((Bb, N, W), jnp.float32),
        ],
        interpret=_INTERPRET,
    )(seed_features, xyzT,
      p['gh_w1'], p['gh_b1'].reshape(C, 1), p['gh_g1'].reshape(C, 1),
      p['gh_be1'].reshape(C, 1), p['gh_m1'].reshape(C, 1),
      p['gh_v1'].reshape(C, 1), p['gh_w2'].reshape(1, C),
      p['gh_b2'].reshape(1, 1))
    return score, featT


def _fps_kernel(xyz_ref, score_ref, inds_ref):
    Bb = xyz_ref.shape[0]
    R = xyz_ref.shape[2]
    Cc = xyz_ref.shape[3]
    N = R * Cc
    x = xyz_ref[:, 0]
    y = xyz_ref[:, 1]
    z = xyz_ref[:, 2]
    s = score_ref[...]
    idx = (lax.broadcasted_iota(jnp.int32, (Bb, R, Cc), 1) * Cc
           + lax.broadcasted_iota(jnp.int32, (Bb, R, Cc), 2))
    m = s > THRESH
    anym = jnp.any(m, axis=(1, 2), keepdims=True)
    m = jnp.logical_or(m, jnp.logical_not(anym))
    dist0 = jnp.where(m, jnp.float32(1e10), -jnp.inf)
    BIG = jnp.int32(N)
    far0 = jnp.min(jnp.where(m, idx, BIG), axis=(1, 2))
    bofs = lax.iota(jnp.int32, Bb) * N

    def body(i, carry):
        dist, far = carry
        sel = idx == far[:, None, None]
        fx = jnp.sum(jnp.where(sel, x, 0.0), axis=(1, 2))
        fy = jnp.sum(jnp.where(sel, y, 0.0), axis=(1, 2))
        fz = jnp.sum(jnp.where(sel, z, 0.0), axis=(1, 2))
        inds_ref[pl.ds(i, 1), :] = jnp.concatenate(
            [far + bofs, jnp.zeros((Bb,), jnp.int32)])[None, :]
        dx = x - fx[:, None, None]
        dy = y - fy[:, None, None]
        dz = z - fz[:, None, None]
        d = dx * dx + dy * dy + dz * dz
        dist = jnp.minimum(dist, d)
        mx = jnp.max(dist, axis=(1, 2))
        far = jnp.min(
            jnp.where(dist == mx[:, None, None], idx, BIG), axis=(1, 2))
        return dist, far

    def body8(i, c):
        for k in range(8):
            c = body(8 * i + k, c)
        return c

    lax.fori_loop(0, NUM_SAMPLE // 8, body8, (dist0, far0))


def _fps(xyzT, score):
    Bb = xyzT.shape[0]
    N = xyzT.shape[2]
    R = 128
    Cc = N // R
    xyz4 = xyzT.reshape(Bb, 3, R, Cc)
    score4 = score.reshape(Bb, R, Cc)
    full = lambda shp: pl.BlockSpec(shp, lambda: (0,) * len(shp))
    inds = pl.pallas_call(
        _fps_kernel,
        in_specs=[full((Bb, 3, R, Cc)), full((Bb, R, Cc))],
        out_specs=full((NUM_SAMPLE, 2 * Bb)),
        out_shape=jax.ShapeDtypeStruct((NUM_SAMPLE, 2 * Bb), jnp.int32),
        interpret=_INTERPRET,
    )(xyz4, score4)
    return inds[:, :Bb].T.reshape(-1)


def _gather_rows(featT_flat, inds_flat):
    M, D = featT_flat.shape
    T = inds_flat.shape[0]
    NW = 32
    per = T // NW
    mesh = plsc.VectorSubcoreMesh(core_axis_name="c", subcore_axis_name="s")

    @functools.partial(
        pl.kernel, mesh=mesh,
        out_type=jax.ShapeDtypeStruct((T, D), jnp.float32),
        scratch_types=[
            pltpu.VMEM((per,), jnp.int32),
            pltpu.VMEM((per, D), jnp.float32),
            pltpu.SemaphoreType.DMA,
        ],
    )
    def gk(inds_hbm, tab_hbm, out_hbm, idx_v, rows_v, sem):
        wid = lax.axis_index("s") * 2 + lax.axis_index("c")
        base = wid * per
        pltpu.sync_copy(inds_hbm.at[pl.ds(base, per)], idx_v)
        pltpu.async_copy(tab_hbm.at[idx_v], rows_v, sem).wait()
        pltpu.sync_copy(rows_v, out_hbm.at[pl.ds(base, per)])

    return gk(inds_flat, featT_flat)


def _view_kernel(f_ref, w1_ref, b1_ref, g1_ref, be1_ref, m1_ref, v1_ref,
                 w2_ref, b2_ref, g2_ref, be2_ref, m2_ref, v2_ref,
                 w3_ref, b3_ref, tvx_ref, tvy_ref, tvz_ref,
                 vs_ref, tvs_ref, vp_ref, rot_ref):
    F = f_ref[0]
    h = jnp.dot(w1_ref[...], F, preferred_element_type=jnp.float32) + b1_ref[...]
    h = (h - m1_ref[...]) / jnp.sqrt(v1_ref[...] + EPS) * g1_ref[...] + be1_ref[...]
    h = jnp.maximum(h, 0.0)
    h = jnp.dot(w2_ref[...], h, preferred_element_type=jnp.float32) + b2_ref[...]
    h = (h - m2_ref[...]) / jnp.sqrt(v2_ref[...] + EPS) * g2_ref[...] + be2_ref[...]
    h = jnp.maximum(h, 0.0)
    h3 = jnp.dot(w3_ref[...], h, preferred_element_type=jnp.float32) + b3_ref[...]
    vs_ref[0] = h3

    V, NS = h3.shape
    rowidx = lax.broadcasted_iota(jnp.int32, (V, NS), 0)
    h3m = jnp.where(rowidx < NUM_VIEW, h3, -jnp.inf)
    mx = jnp.max(h3m, axis=0, keepdims=True)
    tvs_ref[0] = mx
    am = jnp.min(jnp.where(h3m == mx, rowidx, jnp.int32(V)), axis=0,
                 keepdims=True)
    sel = rowidx == am
    vpx = jnp.sum(jnp.where(sel, tvx_ref[...], 0.0), axis=0, keepdims=True)
    vpy = jnp.sum(jnp.where(sel, tvy_ref[...], 0.0), axis=0, keepdims=True)
    vpz = jnp.sum(jnp.where(sel, tvz_ref[...], 0.0), axis=0, keepdims=True)
    vp_ref[0] = jnp.concatenate([vpx, vpy, vpz], axis=0)

    tx, ty, tz = -vpx, -vpy, -vpz
    ay0r, ay1r = -ty, tx
    ny = jnp.sqrt(ay0r * ay0r + ay1r * ay1r + 0.0)
    mz = ny == 0.0
    ay0 = jnp.where(mz, 0.0, ay0r)
    ay1 = jnp.where(mz, 1.0, ay1r)
    ay2 = jnp.zeros_like(ay0)
    nx = jnp.sqrt(tx * tx + ty * ty + tz * tz)
    ax0, ax1, ax2 = tx / nx, ty / nx, tz / nx
    nyn = jnp.sqrt(ay0 * ay0 + ay1 * ay1 + ay2 * ay2)
    ay0, ay1, ay2 = ay0 / nyn, ay1 / nyn, ay2 / nyn
    az0 = ax1 * ay2 - ax2 * ay1
    az1 = ax2 * ay0 - ax0 * ay2
    az2 = ax0 * ay1 - ax1 * ay0
    rot_ref[0] = jnp.concatenate(
        [ax0, ay0, az0, ax1, ay1, az1, ax2, ay2, az2], axis=0)


def _view_head(gfeat, p, tv_pad):
    Bb, C, NS = gfeat.shape
    V = NUM_VIEW_PAD
    full = lambda shp: pl.BlockSpec(shp, lambda b: (0,) * len(shp))

    def padw(w):
        return jnp.concatenate(
            [w, jnp.zeros((V - NUM_VIEW, w.shape[1]), w.dtype)], axis=0)

    def padv(v, fill=0.0):
        return jnp.concatenate(
            [v, jnp.full((V - NUM_VIEW,), fill, v.dtype)]).reshape(V, 1)

    vs, tvs, vp, rot = pl.pallas_call(
        _view_kernel,
        grid=(Bb,),
        in_specs=[
            pl.BlockSpec((1, C, NS), lambda b: (b, 0, 0)),
            full((C, C)), full((C, 1)), full((C, 1)), full((C, 1)),
            full((C, 1)), full((C, 1)),
            full((V, C)), full((V, 1)), full((V, 1)), full((V, 1)),
            full((V, 1)), full((V, 1)),
            full((V, V)), full((V, 1)),
            full((V, 1)), full((V, 1)), full((V, 1)),
        ],
        out_specs=[
            pl.BlockSpec((1, V, NS), lambda b: (b, 0, 0)),
            pl.BlockSpec((1, 1, NS), lambda b: (b, 0, 0)),
            pl.BlockSpec((1, 3, NS), lambda b: (b, 0, 0)),
            pl.BlockSpec((1, 9, NS), lambda b: (b, 0, 0)),
        ],
        out_shape=[
            jax.ShapeDtypeStruct((Bb, V, NS), jnp.float32),
            jax.ShapeDtypeStruct((Bb, 1, NS), jnp.float32),
            jax.ShapeDtypeStruct((Bb, 3, NS), jnp.float32),
            jax.ShapeDtypeStruct((Bb, 9, NS), jnp.float32),
        ],
        interpret=_INTERPRET,
    )(gfeat,
      p['w1'], p['b1'].reshape(C, 1), p['g1'].reshape(C, 1),
      p['be1'].reshape(C, 1), p['m1'].reshape(C, 1), p['v1'].reshape(C, 1),
      padw(p['w2']), padv(p['b2']), padv(p['g2']), padv(p['be2']),
      padv(p['m2']), padv(p['v2'], 1.0),
      jnp.pad(padw(p['w3']), ((0, 0), (0, V - NUM_VIEW))), padv(p['b3']),
      tv_pad[:, 0].reshape(V, 1), tv_pad[:, 1].reshape(V, 1),
      tv_pad[:, 2].reshape(V, 1))
    return vs, tvs, vp, rot


def kernel(seed_xyz, seed_features, params):
    Bb, N, _ = seed_xyz.shape
    C = seed_features.shape[1]

    xyzT = seed_xyz.transpose(0, 2, 1)
    score3, featT = _graspness_head(seed_features, xyzT, params)
    graspness_score = score3[:, 0, :]

    inds_g = _fps(xyzT, graspness_score)

    gflat = _gather_rows(featT.reshape(Bb * N, C + 128), inds_g)
    graspable_features = gflat[:, :C].reshape(Bb, NUM_SAMPLE, C).transpose(0, 2, 1)
    graspable_xyz = gflat[:, C:C + 3].reshape(Bb, NUM_SAMPLE, 3)
    fp2_graspness = gflat[:, C + 3].reshape(Bb, NUM_SAMPLE)

    tv = jnp.asarray(np.concatenate(
        [_template_views(NUM_VIEW),
         np.zeros((NUM_VIEW_PAD - NUM_VIEW, 3), np.float32)], axis=0))
    vs, tvs, vp, rot = _view_head(graspable_features, params, tv)

    view_score = vs[:, :NUM_VIEW, :].transpose(0, 2, 1)
    top_view_scores = tvs[:, 0, :]
    vp_xyz = vp.transpose(0, 2, 1)
    vp_rot = rot.reshape(Bb, 3, 3, NUM_SAMPLE).transpose(0, 3, 1, 2)

    return (graspness_score, graspable_xyz, graspable_features,
            fp2_graspness, view_score, top_view_scores, vp_xyz, vp_rot)


# speedup vs baseline: 3.2127x; 1.0132x over previous
import functools

import numpy as np
import jax
import jax.numpy as jnp
from jax import lax
from jax.experimental import pallas as pl
from jax.experimental.pallas import tpu as pltpu
from jax.experimental.pallas import tpu_sc as plsc

NUM_VIEW = 300
NUM_VIEW_PAD = 384
FEAT_DIM = 256
NUM_SAMPLE = 1024
EPS = 1e-5
THRESH = 0.09

_INTERPRET = False


def _template_views(n):
    phi = (np.sqrt(5.0) - 1.0) / 2.0
    i = np.arange(n, dtype=np.float64)
    zi = (2.0 * i + 1.0) / n - 1.0
    r = np.sqrt(np.clip(1.0 - zi * zi, 0.0, None))
    xi = r * np.cos(2.0 * i * np.pi * phi)
    yi = r * np.sin(2.0 * i * np.pi * phi)
    return np.stack([xi, yi, zi], axis=1).astype(np.float32)


def _score_kernel(f_ref, xyzT_ref, w1_ref, b1_ref, g1_ref, be1_ref, m1_ref,
                  v1_ref, w2_ref, b2_ref, score_ref, featT_ref):
    x = f_ref[0]
    h = jnp.dot(w1_ref[...], x, preferred_element_type=jnp.float32) + b1_ref[...]
    h = (h - m1_ref[...]) / jnp.sqrt(v1_ref[...] + EPS) * g1_ref[...] + be1_ref[...]
    h = jnp.maximum(h, 0.0)
    s = jnp.dot(w2_ref[...], h, preferred_element_type=jnp.float32) + b2_ref[0, 0]
    score_ref[0] = s
    wide = jnp.concatenate(
        [x, xyzT_ref[0], s, jnp.zeros((124, x.shape[1]), jnp.float32)], axis=0)
    featT_ref[0] = wide.T


def _graspness_head(seed_features, xyzT, p):
    Bb, C, N = seed_features.shape
    W = C + 128
    BLK = 4096
    grid = (Bb, N // BLK)
    full = lambda shp: pl.BlockSpec(shp, lambda b, j: (0,) * len(shp))
    score, featT = pl.pallas_call(
        _score_kernel,
        grid=grid,
        in_specs=[
            pl.BlockSpec((1, C, BLK), lambda b, j: (b, 0, j)),
            pl.BlockSpec((1, 3, BLK), lambda b, j: (b, 0, j)),
            full((C, C)), full((C, 1)), full((C, 1)), full((C, 1)),
            full((C, 1)), full((C, 1)), full((1, C)), full((1, 1)),
        ],
        out_specs=[
            pl.BlockSpec((1, 1, BLK), lambda b, j: (b, 0, j)),
            pl.BlockSpec((1, BLK, W), lambda b, j: (b, j, 0)),
        ],
        out_shape=[
            jax.ShapeDtypeStruct((Bb, 1, N), jnp.float32),
            jax.ShapeDtypeStruct((Bb, N, W), jnp.float32),
        ],
        interpret=_INTERPRET,
    )(seed_features, xyzT,
      p['gh_w1'], p['gh_b1'].reshape(C, 1), p['gh_g1'].reshape(C, 1),
      p['gh_be1'].reshape(C, 1), p['gh_m1'].reshape(C, 1),
      p['gh_v1'].reshape(C, 1), p['gh_w2'].reshape(1, C),
      p['gh_b2'].reshape(1, 1))
    return score, featT


def _fps_kernel(xyz_ref, score_ref, inds_ref):
    Bb = xyz_ref.shape[0]
    R = xyz_ref.shape[2]
    Cc = xyz_ref.shape[3]
    N = R * Cc
    x = xyz_ref[:, 0]
    y = xyz_ref[:, 1]
    z = xyz_ref[:, 2]
    s = score_ref[...]
    idx = (lax.broadcasted_iota(jnp.int32, (Bb, R, Cc), 1) * Cc
           + lax.broadcasted_iota(jnp.int32, (Bb, R, Cc), 2))
    m = s > THRESH
    anym = jnp.any(m, axis=(1, 2), keepdims=True)
    m = jnp.logical_or(m, jnp.logical_not(anym))
    dist0 = jnp.where(m, jnp.float32(1e10), -jnp.inf)
    BIG = jnp.int32(N)
    far0 = jnp.min(jnp.where(m, idx, BIG), axis=(1, 2))
    bofs = lax.iota(jnp.int32, Bb) * N

    def body(i, carry):
        dist, far = carry
        sel = idx == far[:, None, None]
        fx = jnp.sum(jnp.where(sel, x, 0.0), axis=(1, 2))
        fy = jnp.sum(jnp.where(sel, y, 0.0), axis=(1, 2))
        fz = jnp.sum(jnp.where(sel, z, 0.0), axis=(1, 2))
        inds_ref[pl.ds(i, 1), :] = jnp.concatenate(
            [far + bofs, jnp.zeros((Bb,), jnp.int32)])[None, :]
        dx = x - fx[:, None, None]
        dy = y - fy[:, None, None]
        dz = z - fz[:, None, None]
        d = dx * dx + dy * dy + dz * dz
        dist = jnp.minimum(dist, d)
        mx = jnp.max(dist, axis=(1, 2))
        far = jnp.min(
            jnp.where(dist == mx[:, None, None], idx, BIG), axis=(1, 2))
        return dist, far

    def body8(i, c):
        for k in range(8):
            c = body(8 * i + k, c)
        return c

    lax.fori_loop(0, NUM_SAMPLE // 8, body8, (dist0, far0))


def _fps(xyzT, score):
    Bb = xyzT.shape[0]
    N = xyzT.shape[2]
    R = 128
    Cc = N // R
    xyz4 = xyzT.reshape(Bb, 3, R, Cc)
    score4 = score.reshape(Bb, R, Cc)
    full = lambda shp: pl.BlockSpec(shp, lambda: (0,) * len(shp))
    inds = pl.pallas_call(
        _fps_kernel,
        in_specs=[full((Bb, 3, R, Cc)), full((Bb, R, Cc))],
        out_specs=full((NUM_SAMPLE, 2 * Bb)),
        out_shape=jax.ShapeDtypeStruct((NUM_SAMPLE, 2 * Bb), jnp.int32),
        interpret=_INTERPRET,
    )(xyz4, score4)
    return inds[:, :Bb].T.reshape(-1)


def _gather_rows(featT_flat, inds_flat):
    M, D = featT_flat.shape
    T = inds_flat.shape[0]
    NW = 32
    per = T // NW
    mesh = plsc.VectorSubcoreMesh(core_axis_name="c", subcore_axis_name="s")

    @functools.partial(
        pl.kernel, mesh=mesh,
        out_type=jax.ShapeDtypeStruct((T, D), jnp.float32),
        scratch_types=[
            pltpu.VMEM((per,), jnp.int32),
            pltpu.VMEM((per, D), jnp.float32),
            pltpu.SemaphoreType.DMA,
        ],
    )
    def gk(inds_hbm, tab_hbm, out_hbm, idx_v, rows_v, sem):
        wid = lax.axis_index("s") * 2 + lax.axis_index("c")
        base = wid * per
        pltpu.sync_copy(inds_hbm.at[pl.ds(base, per)], idx_v)
        pltpu.async_copy(tab_hbm.at[idx_v], rows_v, sem).wait()
        pltpu.sync_copy(rows_v, out_hbm.at[pl.ds(base, per)])

    return gk(inds_flat, featT_flat)


def _view_kernel(f_ref, w1_ref, b1_ref, g1_ref, be1_ref, m1_ref, v1_ref,
                 w2_ref, b2_ref, g2_ref, be2_ref, m2_ref, v2_ref,
                 w3_ref, b3_ref, tvx_ref, tvy_ref, tvz_ref,
                 vs_ref, tvs_ref, vp_ref, rot_ref):
    F = f_ref[0]
    h = jnp.dot(w1_ref[...], F, preferred_element_type=jnp.float32) + b1_ref[...]
    h = (h - m1_ref[...]) / jnp.sqrt(v1_ref[...] + EPS) * g1_ref[...] + be1_ref[...]
    h = jnp.maximum(h, 0.0)
    h = jnp.dot(w2_ref[...], h, preferred_element_type=jnp.float32) + b2_ref[...]
    h = (h - m2_ref[...]) / jnp.sqrt(v2_ref[...] + EPS) * g2_ref[...] + be2_ref[...]
    h = jnp.maximum(h, 0.0)
    h3 = jnp.dot(w3_ref[...], h, preferred_element_type=jnp.float32) + b3_ref[...]
    vs_ref[0] = h3

    V, NS = h3.shape
    rowidx = lax.broadcasted_iota(jnp.int32, (V, NS), 0)
    h3m = jnp.where(rowidx < NUM_VIEW, h3, -jnp.inf)
    mx = jnp.max(h3m, axis=0, keepdims=True)
    tvs_ref[0] = mx
    am = jnp.min(jnp.where(h3m == mx, rowidx, jnp.int32(V)), axis=0,
                 keepdims=True)
    sel = rowidx == am
    vpx = jnp.sum(jnp.where(sel, tvx_ref[...], 0.0), axis=0, keepdims=True)
    vpy = jnp.sum(jnp.where(sel, tvy_ref[...], 0.0), axis=0, keepdims=True)
    vpz = jnp.sum(jnp.where(sel, tvz_ref[...], 0.0), axis=0, keepdims=True)
    vp_ref[0] = jnp.concatenate([vpx, vpy, vpz], axis=0)

    tx, ty, tz = -vpx, -vpy, -vpz
    ay0r, ay1r = -ty, tx
    ny = jnp.sqrt(ay0r * ay0r + ay1r * ay1r + 0.0)
    mz = ny == 0.0
    ay0 = jnp.where(mz, 0.0, ay0r)
    ay1 = jnp.where(mz, 1.0, ay1r)
    ay2 = jnp.zeros_like(ay0)
    nx = jnp.sqrt(tx * tx + ty * ty + tz * tz)
    ax0, ax1, ax2 = tx / nx, ty / nx, tz / nx
    nyn = jnp.sqrt(ay0 * ay0 + ay1 * ay1 + ay2 * ay2)
    ay0, ay1, ay2 = ay0 / nyn, ay1 / nyn, ay2 / nyn
    az0 = ax1 * ay2 - ax2 * ay1
    az1 = ax2 * ay0 - ax0 * ay2
    az2 = ax0 * ay1 - ax1 * ay0
    rot_ref[0] = jnp.concatenate(
        [ax0, ay0, az0, ax1, ay1, az1, ax2, ay2, az2], axis=0)


def _view_head(gfeat, p, tv_pad):
    Bb, C, NS = gfeat.shape
    V = NUM_VIEW_PAD
    full = lambda shp: pl.BlockSpec(shp, lambda b: (0,) * len(shp))

    def padw(w):
        return jnp.concatenate(
            [w, jnp.zeros((V - NUM_VIEW, w.shape[1]), w.dtype)], axis=0)

    def padv(v, fill=0.0):
        return jnp.concatenate(
            [v, jnp.full((V - NUM_VIEW,), fill, v.dtype)]).reshape(V, 1)

    vs, tvs, vp, rot = pl.pallas_call(
        _view_kernel,
        grid=(Bb,),
        in_specs=[
            pl.BlockSpec((1, C, NS), lambda b: (b, 0, 0)),
            full((C, C)), full((C, 1)), full((C, 1)), full((C, 1)),
            full((C, 1)), full((C, 1)),
            full((V, C)), full((V, 1)), full((V, 1)), full((V, 1)),
            full((V, 1)), full((V, 1)),
            full((V, V)), full((V, 1)),
            full((V, 1)), full((V, 1)), full((V, 1)),
        ],
        out_specs=[
            pl.BlockSpec((1, V, NS), lambda b: (b, 0, 0)),
            pl.BlockSpec((1, 1, NS), lambda b: (b, 0, 0)),
            pl.BlockSpec((1, 3, NS), lambda b: (b, 0, 0)),
            pl.BlockSpec((1, 9, NS), lambda b: (b, 0, 0)),
        ],
        out_shape=[
            jax.ShapeDtypeStruct((Bb, V, NS), jnp.float32),
            jax.ShapeDtypeStruct((Bb, 1, NS), jnp.float32),
            jax.ShapeDtypeStruct((Bb, 3, NS), jnp.float32),
            jax.ShapeDtypeStruct((Bb, 9, NS), jnp.float32),
        ],
        interpret=_INTERPRET,
    )(gfeat,
      p['w1'], p['b1'].reshape(C, 1), p['g1'].reshape(C, 1),
      p['be1'].reshape(C, 1), p['m1'].reshape(C, 1), p['v1'].reshape(C, 1),
      padw(p['w2']), padv(p['b2']), padv(p['g2']), padv(p['be2']),
      padv(p['m2']), padv(p['v2'], 1.0),
      jnp.pad(padw(p['w3']), ((0, 0), (0, V - NUM_VIEW))), padv(p['b3']),
      tv_pad[:, 0].reshape(V, 1), tv_pad[:, 1].reshape(V, 1),
      tv_pad[:, 2].reshape(V, 1))
    return vs, tvs, vp, rot


def kernel(seed_xyz, seed_features, params):
    Bb, N, _ = seed_xyz.shape
    C = seed_features.shape[1]

    xyzT = seed_xyz.transpose(0, 2, 1)
    score3, featT = _graspness_head(seed_features, xyzT, params)
    graspness_score = score3[:, 0, :]

    inds_g = _fps(xyzT, graspness_score)

    gflat = _gather_rows(featT.reshape(Bb * N, C + 128), inds_g)
    graspable_features = gflat[:, :C].reshape(Bb, NUM_SAMPLE, C).transpose(0, 2, 1)
    graspable_xyz = gflat[:, C:C + 3].reshape(Bb, NUM_SAMPLE, 3)
    fp2_graspness = gflat[:, C + 3].reshape(Bb, NUM_SAMPLE)

    tv = jnp.asarray(np.concatenate(
        [_template_views(NUM_VIEW),
         np.zeros((NUM_VIEW_PAD - NUM_VIEW, 3), np.float32)], axis=0))
    vs, tvs, vp, rot = _view_head(graspable_features, params, tv)

    view_score = vs[:, :NUM_VIEW, :].transpose(0, 2, 1)
    top_view_scores = tvs[:, 0, :]
    vp_xyz = vp.transpose(0, 2, 1)
    vp_rot = rot.reshape(Bb, 3, 3, NUM_SAMPLE).transpose(0, 3, 1, 2)

    return (graspness_score, graspable_xyz, graspable_features,
            fp2_graspness, view_score, top_view_scores, vp_xyz, vp_rot)


# speedup vs baseline: 3.2180x; 1.0017x over previous
import functools

import numpy as np
import jax
import jax.numpy as jnp
from jax import lax
from jax.experimental import pallas as pl
from jax.experimental.pallas import tpu as pltpu
from jax.experimental.pallas import tpu_sc as plsc

NUM_VIEW = 300
NUM_VIEW_PAD = 384
FEAT_DIM = 256
NUM_SAMPLE = 1024
EPS = 1e-5
THRESH = 0.09


def _template_views(n):
    phi = (np.sqrt(5.0) - 1.0) / 2.0
    i = np.arange(n, dtype=np.float64)
    zi = (2.0 * i + 1.0) / n - 1.0
    r = np.sqrt(np.clip(1.0 - zi * zi, 0.0, None))
    xi = r * np.cos(2.0 * i * np.pi * phi)
    yi = r * np.sin(2.0 * i * np.pi * phi)
    return np.stack([xi, yi, zi], axis=1).astype(np.float32)


def _score_kernel(f_ref, xyzT_ref, w1_ref, b1_ref, g1_ref, be1_ref, m1_ref,
                  v1_ref, w2_ref, b2_ref, score_ref, featT_ref):
    x = f_ref[0]
    h = jnp.dot(w1_ref[...], x, preferred_element_type=jnp.float32) + b1_ref[...]
    h = (h - m1_ref[...]) / jnp.sqrt(v1_ref[...] + EPS) * g1_ref[...] + be1_ref[...]
    h = jnp.maximum(h, 0.0)
    s = jnp.dot(w2_ref[...], h, preferred_element_type=jnp.float32) + b2_ref[0, 0]
    score_ref[0] = s
    wide = jnp.concatenate(
        [x, xyzT_ref[0], s, jnp.zeros((124, x.shape[1]), jnp.float32)], axis=0)
    featT_ref[0] = wide.T


def _graspness_head(seed_features, xyzT, p):
    Bb, C, N = seed_features.shape
    W = C + 128
    BLK = 4096
    grid = (Bb, N // BLK)
    full = lambda shp: pl.BlockSpec(shp, lambda b, j: (0,) * len(shp))
    score, featT = pl.pallas_call(
        _score_kernel,
        grid=grid,
        in_specs=[
            pl.BlockSpec((1, C, BLK), lambda b, j: (b, 0, j)),
            pl.BlockSpec((1, 3, BLK), lambda b, j: (b, 0, j)),
            full((C, C)), full((C, 1)), full((C, 1)), full((C, 1)),
            full((C, 1)), full((C, 1)), full((1, C)), full((1, 1)),
        ],
        out_specs=[
            pl.BlockSpec((1, 1, BLK), lambda b, j: (b, 0, j)),
            pl.BlockSpec((1, BLK, W), lambda b, j: (b, j, 0)),
        ],
        out_shape=[
            jax.ShapeDtypeStruct((Bb, 1, N), jnp.float32),
            jax.ShapeDtypeStruct((Bb, N, W), jnp.float32),
        ],
    )(seed_features, xyzT,
      p['gh_w1'], p['gh_b1'].reshape(C, 1), p['gh_g1'].reshape(C, 1),
      p['gh_be1'].reshape(C, 1), p['gh_m1'].reshape(C, 1),
      p['gh_v1'].reshape(C, 1), p['gh_w2'].reshape(1, C),
      p['gh_b2'].reshape(1, 1))
    return score, featT


def _fps_kernel(xyz_ref, score_ref, inds_ref):
    Bb = xyz_ref.shape[0]
    R = xyz_ref.shape[2]
    Cc = xyz_ref.shape[3]
    N = R * Cc
    x = xyz_ref[:, 0]
    y = xyz_ref[:, 1]
    z = xyz_ref[:, 2]
    s = score_ref[...]
    idx = (lax.broadcasted_iota(jnp.int32, (Bb, R, Cc), 1) * Cc
           + lax.broadcasted_iota(jnp.int32, (Bb, R, Cc), 2))
    m = s > THRESH
    anym = jnp.any(m, axis=(1, 2), keepdims=True)
    m = jnp.logical_or(m, jnp.logical_not(anym))
    dist0 = jnp.where(m, jnp.float32(1e10), -jnp.inf)
    BIG = jnp.int32(N)
    far0 = jnp.min(jnp.where(m, idx, BIG), axis=(1, 2))
    bofs = lax.iota(jnp.int32, Bb) * N

    def body(i, carry):
        dist, far = carry
        sel = idx == far[:, None, None]
        fx = jnp.sum(jnp.where(sel, x, 0.0), axis=(1, 2))
        fy = jnp.sum(jnp.where(sel, y, 0.0), axis=(1, 2))
        fz = jnp.sum(jnp.where(sel, z, 0.0), axis=(1, 2))
        inds_ref[pl.ds(i, 1), :] = jnp.concatenate(
            [far + bofs, jnp.zeros((Bb,), jnp.int32)])[None, :]
        dx = x - fx[:, None, None]
        dy = y - fy[:, None, None]
        dz = z - fz[:, None, None]
        d = dx * dx + dy * dy + dz * dz
        dist = jnp.minimum(dist, d)
        mx = jnp.max(dist, axis=(1, 2))
        far = jnp.min(
            jnp.where(dist == mx[:, None, None], idx, BIG), axis=(1, 2))
        return dist, far

    def body8(i, c):
        for k in range(8):
            c = body(8 * i + k, c)
        return c

    lax.fori_loop(0, NUM_SAMPLE // 8, body8, (dist0, far0))


def _fps(xyzT, score):
    Bb = xyzT.shape[0]
    N = xyzT.shape[2]
    R = 128
    Cc = N // R
    xyz4 = xyzT.reshape(Bb, 3, R, Cc)
    score4 = score.reshape(Bb, R, Cc)
    full = lambda shp: pl.BlockSpec(shp, lambda: (0,) * len(shp))
    inds = pl.pallas_call(
        _fps_kernel,
        in_specs=[full((Bb, 3, R, Cc)), full((Bb, R, Cc))],
        out_specs=full((NUM_SAMPLE, 2 * Bb)),
        out_shape=jax.ShapeDtypeStruct((NUM_SAMPLE, 2 * Bb), jnp.int32),
    )(xyz4, score4)
    return inds[:, :Bb].T.reshape(-1)


def _gather_rows(featT_flat, inds_flat):
    M, D = featT_flat.shape
    T = inds_flat.shape[0]
    NW = 32
    per = T // NW
    mesh = plsc.VectorSubcoreMesh(core_axis_name="c", subcore_axis_name="s")

    @functools.partial(
        pl.kernel, mesh=mesh,
        out_type=jax.ShapeDtypeStruct((T, D), jnp.float32),
        scratch_types=[
            pltpu.VMEM((per,), jnp.int32),
            pltpu.VMEM((per, D), jnp.float32),
            pltpu.SemaphoreType.DMA,
        ],
    )
    def gk(inds_hbm, tab_hbm, out_hbm, idx_v, rows_v, sem):
        wid = lax.axis_index("s") * 2 + lax.axis_index("c")
        base = wid * per
        pltpu.sync_copy(inds_hbm.at[pl.ds(base, per)], idx_v)
        pltpu.async_copy(tab_hbm.at[idx_v], rows_v, sem).wait()
        pltpu.sync_copy(rows_v, out_hbm.at[pl.ds(base, per)])

    return gk(inds_flat, featT_flat)


def _view_kernel(f_ref, w1_ref, b1_ref, g1_ref, be1_ref, m1_ref, v1_ref,
                 w2_ref, b2_ref, g2_ref, be2_ref, m2_ref, v2_ref,
                 w3_ref, b3_ref, tvx_ref, tvy_ref, tvz_ref,
                 vs_ref, tvs_ref, vp_ref, rot_ref):
    F = f_ref[0]
    h = jnp.dot(w1_ref[...], F, preferred_element_type=jnp.float32) + b1_ref[...]
    h = (h - m1_ref[...]) / jnp.sqrt(v1_ref[...] + EPS) * g1_ref[...] + be1_ref[...]
    h = jnp.maximum(h, 0.0)
    h = jnp.dot(w2_ref[...], h, preferred_element_type=jnp.float32) + b2_ref[...]
    h = (h - m2_ref[...]) / jnp.sqrt(v2_ref[...] + EPS) * g2_ref[...] + be2_ref[...]
    h = jnp.maximum(h, 0.0)
    h3 = jnp.dot(w3_ref[...], h, preferred_element_type=jnp.float32) + b3_ref[...]
    vs_ref[0] = h3

    V, NS = h3.shape
    rowidx = lax.broadcasted_iota(jnp.int32, (V, NS), 0)
    h3m = jnp.where(rowidx < NUM_VIEW, h3, -jnp.inf)
    mx = jnp.max(h3m, axis=0, keepdims=True)
    tvs_ref[0] = mx
    am = jnp.min(jnp.where(h3m == mx, rowidx, jnp.int32(V)), axis=0,
                 keepdims=True)
    sel = rowidx == am
    vpx = jnp.sum(jnp.where(sel, tvx_ref[...], 0.0), axis=0, keepdims=True)
    vpy = jnp.sum(jnp.where(sel, tvy_ref[...], 0.0), axis=0, keepdims=True)
    vpz = jnp.sum(jnp.where(sel, tvz_ref[...], 0.0), axis=0, keepdims=True)
    vp_ref[0] = jnp.concatenate([vpx, vpy, vpz], axis=0)

    tx, ty, tz = -vpx, -vpy, -vpz
    ay0r, ay1r = -ty, tx
    ny = jnp.sqrt(ay0r * ay0r + ay1r * ay1r + 0.0)
    mz = ny == 0.0
    ay0 = jnp.where(mz, 0.0, ay0r)
    ay1 = jnp.where(mz, 1.0, ay1r)
    ay2 = jnp.zeros_like(ay0)
    nx = jnp.sqrt(tx * tx + ty * ty + tz * tz)
    ax0, ax1, ax2 = tx / nx, ty / nx, tz / nx
    nyn = jnp.sqrt(ay0 * ay0 + ay1 * ay1 + ay2 * ay2)
    ay0, ay1, ay2 = ay0 / nyn, ay1 / nyn, ay2 / nyn
    az0 = ax1 * ay2 - ax2 * ay1
    az1 = ax2 * ay0 - ax0 * ay2
    az2 = ax0 * ay1 - ax1 * ay0
    rot_ref[0] = jnp.concatenate(
        [ax0, ay0, az0, ax1, ay1, az1, ax2, ay2, az2], axis=0)


def _view_head(gfeat, p, tv_pad):
    Bb, C, NS = gfeat.shape
    V = NUM_VIEW_PAD
    full = lambda shp: pl.BlockSpec(shp, lambda b: (0,) * len(shp))

    def padw(w):
        return jnp.concatenate(
            [w, jnp.zeros((V - NUM_VIEW, w.shape[1]), w.dtype)], axis=0)

    def padv(v, fill=0.0):
        return jnp.concatenate(
            [v, jnp.full((V - NUM_VIEW,), fill, v.dtype)]).reshape(V, 1)

    vs, tvs, vp, rot = pl.pallas_call(
        _view_kernel,
        grid=(Bb,),
        in_specs=[
            pl.BlockSpec((1, C, NS), lambda b: (b, 0, 0)),
            full((C, C)), full((C, 1)), full((C, 1)), full((C, 1)),
            full((C, 1)), full((C, 1)),
            full((V, C)), full((V, 1)), full((V, 1)), full((V, 1)),
            full((V, 1)), full((V, 1)),
            full((V, V)), full((V, 1)),
            full((V, 1)), full((V, 1)), full((V, 1)),
        ],
        out_specs=[
            pl.BlockSpec((1, V, NS), lambda b: (b, 0, 0)),
            pl.BlockSpec((1, 1, NS), lambda b: (b, 0, 0)),
            pl.BlockSpec((1, 3, NS), lambda b: (b, 0, 0)),
            pl.BlockSpec((1, 9, NS), lambda b: (b, 0, 0)),
        ],
        out_shape=[
            jax.ShapeDtypeStruct((Bb, V, NS), jnp.float32),
            jax.ShapeDtypeStruct((Bb, 1, NS), jnp.float32),
            jax.ShapeDtypeStruct((Bb, 3, NS), jnp.float32),
            jax.ShapeDtypeStruct((Bb, 9, NS), jnp.float32),
        ],
    )(gfeat,
      p['w1'], p['b1'].reshape(C, 1), p['g1'].reshape(C, 1),
      p['be1'].reshape(C, 1), p['m1'].reshape(C, 1), p['v1'].reshape(C, 1),
      padw(p['w2']), padv(p['b2']), padv(p['g2']), padv(p['be2']),
      padv(p['m2']), padv(p['v2'], 1.0),
      jnp.pad(padw(p['w3']), ((0, 0), (0, V - NUM_VIEW))), padv(p['b3']),
      tv_pad[:, 0].reshape(V, 1), tv_pad[:, 1].reshape(V, 1),
      tv_pad[:, 2].reshape(V, 1))
    return vs, tvs, vp, rot


def kernel(seed_xyz, seed_features, params):
    Bb, N, _ = seed_xyz.shape
    C = seed_features.shape[1]

    xyzT = seed_xyz.transpose(0, 2, 1)
    score3, featT = _graspness_head(seed_features, xyzT, params)
    graspness_score = score3[:, 0, :]

    inds_g = _fps(xyzT, graspness_score)

    gflat = _gather_rows(featT.reshape(Bb * N, C + 128), inds_g)
    graspable_features = gflat[:, :C].reshape(Bb, NUM_SAMPLE, C).transpose(0, 2, 1)
    graspable_xyz = gflat[:, C:C + 3].reshape(Bb, NUM_SAMPLE, 3)
    fp2_graspness = gflat[:, C + 3].reshape(Bb, NUM_SAMPLE)

    tv = jnp.asarray(np.concatenate(
        [_template_views(NUM_VIEW),
         np.zeros((NUM_VIEW_PAD - NUM_VIEW, 3), np.float32)], axis=0))
    vs, tvs, vp, rot = _view_head(graspable_features, params, tv)

    view_score = vs[:, :NUM_VIEW, :].transpose(0, 2, 1)
    top_view_scores = tvs[:, 0, :]
    vp_xyz = vp.transpose(0, 2, 1)
    vp_rot = rot.reshape(Bb, 3, 3, NUM_SAMPLE).transpose(0, 3, 1, 2)

    return (graspness_score, graspable_xyz, graspable_features,
            fp2_graspness, view_score, top_view_scores, vp_xyz, vp_rot)
